# Initial kernel scaffold; baseline (speedup 1.0000x reference)
import jax
import jax.numpy as jnp
from jax.experimental import pallas as pl


def kernel(x, table, W, b):
    raise NotImplementedError("write your pallas kernel here")


# speedup vs baseline: 2.2474x; 2.2474x over previous
import functools

import jax
import jax.numpy as jnp
from jax import lax
from jax.experimental import pallas as pl
from jax.experimental.pallas import tpu as pltpu
from jax.experimental.pallas import tpu_sc as plsc

VOCAB = 1000000
EMBED_DIM = 64
B = 16384
L = 50
N = B * L


TC_ROWS = 8000


def _precompute_body(tab_ref, w_ref, b_ref, out_ref):
    t = tab_ref[...]
    w = w_ref[...]
    z = jnp.dot(t, w, preferred_element_type=jnp.float32) + b_ref[0, 0]
    out_ref[...] = jax.nn.sigmoid(z)


def _precompute(table, W, b):
    return pl.pallas_call(
        _precompute_body,
        grid=(VOCAB // TC_ROWS,),
        in_specs=[
            pl.BlockSpec((TC_ROWS, EMBED_DIM), lambda i: (i, 0)),
            pl.BlockSpec((EMBED_DIM, 1), lambda i: (0, 0)),
            pl.BlockSpec(memory_space=pltpu.SMEM),
        ],
        out_specs=pl.BlockSpec((TC_ROWS, 1), lambda i: (i, 0)),
        out_shape=jax.ShapeDtypeStruct((VOCAB, 1), jnp.float32),
    )(table, W, b.reshape(1, 1))


_NC, _NS = 2, 16
_NW = _NC * _NS
_PER_W = N // _NW


def _gather_body(v_hbm, idx_hbm, out_hbm, idx_v, val_v, sem):
    wid = lax.axis_index("s") * _NC + lax.axis_index("c")
    base = wid * _PER_W
    pltpu.sync_copy(idx_hbm.at[pl.ds(base, _PER_W)], idx_v)
    pltpu.async_copy(v_hbm.at[idx_v], val_v, sem).wait()
    pltpu.sync_copy(val_v, out_hbm.at[pl.ds(base, _PER_W)])


def _sc_gather(v_flat, idx_flat):
    mesh = plsc.VectorSubcoreMesh(core_axis_name="c", subcore_axis_name="s")
    return pl.kernel(
        _gather_body,
        mesh=mesh,
        out_type=jax.ShapeDtypeStruct((N,), jnp.float32),
        scratch_types=[
            pltpu.VMEM((_PER_W,), jnp.int32),
            pltpu.VMEM((_PER_W,), jnp.float32),
            pltpu.SemaphoreType.DMA,
        ],
    )(v_flat, idx_flat)


def kernel(x, table, W, b):
    v = _precompute(table, W, b)
    idx = x.reshape(N).astype(jnp.int32)
    out = _sc_gather(v.reshape(VOCAB), idx)
    return out.reshape(B, L, 1)


# speedup vs baseline: 2.4596x; 1.0944x over previous
import functools

import jax
import jax.numpy as jnp
from jax import lax
from jax.experimental import pallas as pl
from jax.experimental.pallas import tpu as pltpu
from jax.experimental.pallas import tpu_sc as plsc

VOCAB = 1000000
EMBED_DIM = 64
B = 16384
L = 50
N = B * L


TC_K = 8
TC_ROWS = 5000
TC_COLS = TC_K * EMBED_DIM


def _precompute_body(tab_ref, w_ref, b_ref, out_ref):
    t = tab_ref[...]
    w = w_ref[...]
    z = jnp.dot(t, w, preferred_element_type=jnp.float32) + b_ref[0, 0]
    out_ref[...] = jax.nn.sigmoid(z)


def _precompute(table, W, b):
    t2 = table.reshape(VOCAB // TC_K, TC_COLS)
    eye = jnp.eye(TC_K, dtype=jnp.float32)
    w_bd = (eye[:, None, :] * W[None, :, 0:1]).reshape(TC_COLS, TC_K)
    return pl.pallas_call(
        _precompute_body,
        grid=(VOCAB // TC_K // TC_ROWS,),
        in_specs=[
            pl.BlockSpec((TC_ROWS, TC_COLS), lambda i: (i, 0)),
            pl.BlockSpec((TC_COLS, TC_K), lambda i: (0, 0)),
            pl.BlockSpec(memory_space=pltpu.SMEM),
        ],
        out_specs=pl.BlockSpec((TC_ROWS, TC_K), lambda i: (i, 0)),
        out_shape=jax.ShapeDtypeStruct((VOCAB // TC_K, TC_K), jnp.float32),
    )(t2, w_bd, b.reshape(1, 1))


_NC, _NS = 2, 16
_NW = _NC * _NS
_PER_W = N // _NW


def _gather_body(v_hbm, idx_hbm, out_hbm, idx_v, val_v, sem):
    wid = lax.axis_index("s") * _NC + lax.axis_index("c")
    base = wid * _PER_W
    pltpu.sync_copy(idx_hbm.at[pl.ds(base, _PER_W)], idx_v)
    pltpu.async_copy(v_hbm.at[idx_v], val_v, sem).wait()
    pltpu.sync_copy(val_v, out_hbm.at[pl.ds(base, _PER_W)])


def _sc_gather(v_flat, idx_flat):
    mesh = plsc.VectorSubcoreMesh(core_axis_name="c", subcore_axis_name="s")
    return pl.kernel(
        _gather_body,
        mesh=mesh,
        out_type=jax.ShapeDtypeStruct((N,), jnp.float32),
        scratch_types=[
            pltpu.VMEM((_PER_W,), jnp.int32),
            pltpu.VMEM((_PER_W,), jnp.float32),
            pltpu.SemaphoreType.DMA,
        ],
    )(v_flat, idx_flat)


def kernel(x, table, W, b):
    v = _precompute(table, W, b)
    idx = x.reshape(N).astype(jnp.int32)
    out = _sc_gather(v.reshape(VOCAB), idx)
    return out.reshape(B, L, 1)


# speedup vs baseline: 2.6216x; 1.0659x over previous
import functools

import jax
import jax.numpy as jnp
from jax import lax
from jax.experimental import pallas as pl
from jax.experimental.pallas import tpu as pltpu
from jax.experimental.pallas import tpu_sc as plsc

VOCAB = 1000000
EMBED_DIM = 64
B = 16384
L = 50
N = B * L


TC_K = 64
TC_ROWS = 1024
TC_COLS = TC_K * EMBED_DIM


def _precompute_body(tab_ref, w_ref, b_ref, out_ref):
    t = tab_ref[...]
    w = w_ref[...]
    z = jnp.dot(t, w, preferred_element_type=jnp.float32) + b_ref[0, 0]
    out_ref[...] = jax.nn.sigmoid(z)


def _precompute(table, W, b):
    t2 = table.reshape(VOCAB // TC_K, TC_COLS)
    eye = jnp.eye(TC_K, dtype=jnp.float32)
    w_bd = (eye[:, None, :] * W[None, :, 0:1]).reshape(TC_COLS, TC_K)
    return pl.pallas_call(
        _precompute_body,
        grid=(pl.cdiv(VOCAB // TC_K, TC_ROWS),),
        in_specs=[
            pl.BlockSpec((TC_ROWS, TC_COLS), lambda i: (i, 0)),
            pl.BlockSpec((TC_COLS, TC_K), lambda i: (0, 0)),
            pl.BlockSpec(memory_space=pltpu.SMEM),
        ],
        out_specs=pl.BlockSpec((TC_ROWS, TC_K), lambda i: (i, 0)),
        out_shape=jax.ShapeDtypeStruct((VOCAB // TC_K, TC_K), jnp.float32),
    )(t2, w_bd, b.reshape(1, 1))


_NC, _NS = 2, 16
_NW = _NC * _NS
_PER_W = N // _NW


def _gather_body(v_hbm, idx_hbm, out_hbm, idx_v, val_v, sem):
    wid = lax.axis_index("s") * _NC + lax.axis_index("c")
    base = wid * _PER_W
    pltpu.sync_copy(idx_hbm.at[pl.ds(base, _PER_W)], idx_v)
    pltpu.async_copy(v_hbm.at[idx_v], val_v, sem).wait()
    pltpu.sync_copy(val_v, out_hbm.at[pl.ds(base, _PER_W)])


def _sc_gather(v_flat, idx_flat):
    mesh = plsc.VectorSubcoreMesh(core_axis_name="c", subcore_axis_name="s")
    return pl.kernel(
        _gather_body,
        mesh=mesh,
        out_type=jax.ShapeDtypeStruct((N,), jnp.float32),
        scratch_types=[
            pltpu.VMEM((_PER_W,), jnp.int32),
            pltpu.VMEM((_PER_W,), jnp.float32),
            pltpu.SemaphoreType.DMA,
        ],
    )(v_flat, idx_flat)


def kernel(x, table, W, b):
    v = _precompute(table, W, b)
    idx = x.reshape(N).astype(jnp.int32)
    out = _sc_gather(v.reshape(VOCAB), idx)
    return out.reshape(B, L, 1)


# speedup vs baseline: 2.6283x; 1.0025x over previous
import functools

import jax
import jax.numpy as jnp
from jax import lax
from jax.experimental import pallas as pl
from jax.experimental.pallas import tpu as pltpu
from jax.experimental.pallas import tpu_sc as plsc

VOCAB = 1000000
VOCAB_PAD = 1024000
EMBED_DIM = 64
B = 16384
L = 50
N = B * L


TC_K = 64
TC_ROWS = 1024
TC_COLS = TC_K * EMBED_DIM


def _precompute_body(tab_ref, w_ref, b_ref, out_ref):
    t = tab_ref[...]
    w = w_ref[...]
    z = jnp.dot(t, w, preferred_element_type=jnp.float32) + b_ref[0, 0]
    s = jax.nn.sigmoid(z)
    h = TC_ROWS // 2
    out_ref[...] = jnp.concatenate([s[:h, :], s[h:, :]], axis=1)


def _precompute(table, W, b):
    t2 = table.reshape(VOCAB // TC_K, TC_COLS)
    eye = jnp.eye(TC_K, dtype=jnp.float32)
    w_bd = (eye[:, None, :] * W[None, :, 0:1]).reshape(TC_COLS, TC_K)
    return pl.pallas_call(
        _precompute_body,
        grid=(pl.cdiv(VOCAB // TC_K, TC_ROWS),),
        in_specs=[
            pl.BlockSpec((TC_ROWS, TC_COLS), lambda i: (i, 0)),
            pl.BlockSpec((TC_COLS, TC_K), lambda i: (0, 0)),
            pl.BlockSpec(memory_space=pltpu.SMEM),
        ],
        out_specs=pl.BlockSpec((TC_ROWS // 2, 2 * TC_K), lambda i: (i, 0)),
        out_shape=jax.ShapeDtypeStruct((VOCAB_PAD // (2 * TC_K), 2 * TC_K),
                                       jnp.float32),
    )(t2, w_bd, b.reshape(1, 1))


_NC, _NS = 2, 16
_NW = _NC * _NS
_PER_W = N // _NW


def _gather_body(v_hbm, idx_hbm, out_hbm, idx_v, val_v, sem):
    wid = lax.axis_index("s") * _NC + lax.axis_index("c")
    base = wid * _PER_W
    pltpu.sync_copy(idx_hbm.at[pl.ds(base, _PER_W)], idx_v)

    def _xform(c, carry):
        t = idx_v[pl.ds(c * 16, 16)]
        p = ((t & -65536)
             | (((t >> 6) & 511) << 7)
             | (((t >> 15) & 1) << 6)
             | (t & 63))
        idx_v[pl.ds(c * 16, 16)] = p
        return carry

    lax.fori_loop(0, _PER_W // 16, _xform, 0)
    pltpu.async_copy(v_hbm.at[idx_v], val_v, sem).wait()
    pltpu.sync_copy(val_v, out_hbm.at[pl.ds(base, _PER_W)])


def _sc_gather(v_flat, idx_flat):
    mesh = plsc.VectorSubcoreMesh(core_axis_name="c", subcore_axis_name="s")
    return pl.kernel(
        _gather_body,
        mesh=mesh,
        out_type=jax.ShapeDtypeStruct((N,), jnp.float32),
        scratch_types=[
            pltpu.VMEM((_PER_W,), jnp.int32),
            pltpu.VMEM((_PER_W,), jnp.float32),
            pltpu.SemaphoreType.DMA,
        ],
    )(v_flat, idx_flat)


def kernel(x, table, W, b):
    v = _precompute(table, W, b)
    idx = x.reshape(N).astype(jnp.int32)
    out = _sc_gather(v.reshape(VOCAB_PAD), idx)
    return out.reshape(B, L, 1)


# speedup vs baseline: 3.3572x; 1.2773x over previous
import functools

import jax
import jax.numpy as jnp
from jax import lax
from jax.experimental import pallas as pl
from jax.experimental.pallas import tpu as pltpu
from jax.experimental.pallas import tpu_sc as plsc

VOCAB = 1000000
EMBED_DIM = 64
B = 16384
L = 50
N = B * L


TC_BS = 8192
TC_GRID = pl.cdiv(VOCAB, TC_BS)
VPAD = TC_GRID * TC_BS


def _precompute_body(tab_ref, w_ref, b_ref, out_ref):
    t = tab_ref[...]
    w = w_ref[...]
    zt = lax.dot_general(w, t, (((1,), (1,)), ((), ())),
                         preferred_element_type=jnp.float32)
    out_ref[...] = jax.nn.sigmoid(zt + b_ref[0, 0]).reshape(TC_BS)


def _precompute(table, W, b):
    return pl.pallas_call(
        _precompute_body,
        grid=(TC_GRID,),
        in_specs=[
            pl.BlockSpec((TC_BS, EMBED_DIM), lambda i: (i, 0)),
            pl.BlockSpec((1, EMBED_DIM), lambda i: (0, 0)),
            pl.BlockSpec(memory_space=pltpu.SMEM),
        ],
        out_specs=pl.BlockSpec((TC_BS,), lambda i: (i,)),
        out_shape=jax.ShapeDtypeStruct((VPAD,), jnp.float32),
    )(table, W.reshape(1, EMBED_DIM), b.reshape(1, 1))


_NC, _NS = 2, 16
_NW = _NC * _NS
_PER_W = N // _NW


def _gather_body(v_hbm, idx_hbm, out_hbm, idx_v, val_v, sem):
    wid = lax.axis_index("s") * _NC + lax.axis_index("c")
    base = wid * _PER_W
    pltpu.sync_copy(idx_hbm.at[pl.ds(base, _PER_W)], idx_v)
    pltpu.async_copy(v_hbm.at[idx_v], val_v, sem).wait()
    pltpu.sync_copy(val_v, out_hbm.at[pl.ds(base, _PER_W)])


def _sc_gather(v_flat, idx_flat):
    mesh = plsc.VectorSubcoreMesh(core_axis_name="c", subcore_axis_name="s")
    return pl.kernel(
        _gather_body,
        mesh=mesh,
        out_type=jax.ShapeDtypeStruct((N,), jnp.float32),
        scratch_types=[
            pltpu.VMEM((_PER_W,), jnp.int32),
            pltpu.VMEM((_PER_W,), jnp.float32),
            pltpu.SemaphoreType.DMA,
        ],
    )(v_flat, idx_flat)


def kernel(x, table, W, b):
    v = _precompute(table, W, b)
    idx = x.reshape(N).astype(jnp.int32)
    out = _sc_gather(v, idx)
    return out.reshape(B, L, 1)


# speedup vs baseline: 12.0878x; 3.6005x over previous
import functools

import jax
import jax.numpy as jnp
from jax import lax
from jax.experimental import pallas as pl
from jax.experimental.pallas import tpu as pltpu
from jax.experimental.pallas import tpu_sc as plsc

VOCAB = 1000000
EMBED_DIM = 64
B = 16384
L = 50
N = B * L


TC_BS = 32768
TC_GRID = pl.cdiv(VOCAB, TC_BS)
VPAD = TC_GRID * TC_BS


def _precompute_body(tab_ref, w_ref, b_ref, out_ref):
    t = tab_ref[...]
    w = w_ref[...]
    zt = jnp.dot(w, t, preferred_element_type=jnp.float32)
    out_ref[...] = jax.nn.sigmoid(zt + b_ref[0, 0]).reshape(TC_BS)


def _precompute(table, W, b):
    t_t = table.T
    return pl.pallas_call(
        _precompute_body,
        grid=(TC_GRID,),
        in_specs=[
            pl.BlockSpec((EMBED_DIM, TC_BS), lambda i: (0, i)),
            pl.BlockSpec((1, EMBED_DIM), lambda i: (0, 0)),
            pl.BlockSpec(memory_space=pltpu.SMEM),
        ],
        out_specs=pl.BlockSpec((TC_BS,), lambda i: (i,)),
        out_shape=jax.ShapeDtypeStruct((VPAD,), jnp.float32),
    )(t_t, W.reshape(1, EMBED_DIM), b.reshape(1, 1))


_NC, _NS = 2, 16
_NW = _NC * _NS
_PER_W = N // _NW


def _gather_body(v_hbm, idx_hbm, out_hbm, idx_v, val_v, sem):
    wid = lax.axis_index("s") * _NC + lax.axis_index("c")
    base = wid * _PER_W
    pltpu.sync_copy(idx_hbm.at[pl.ds(base, _PER_W)], idx_v)
    pltpu.async_copy(v_hbm.at[idx_v], val_v, sem).wait()
    pltpu.sync_copy(val_v, out_hbm.at[pl.ds(base, _PER_W)])


def _sc_gather(v_flat, idx_flat):
    mesh = plsc.VectorSubcoreMesh(core_axis_name="c", subcore_axis_name="s")
    return pl.kernel(
        _gather_body,
        mesh=mesh,
        out_type=jax.ShapeDtypeStruct((N,), jnp.float32),
        scratch_types=[
            pltpu.VMEM((_PER_W,), jnp.int32),
            pltpu.VMEM((_PER_W,), jnp.float32),
            pltpu.SemaphoreType.DMA,
        ],
    )(v_flat, idx_flat)


def kernel(x, table, W, b):
    v = _precompute(table, W, b)
    idx = x.reshape(N).astype(jnp.int32)
    out = _sc_gather(v, idx)
    return out.reshape(B, L, 1)


# speedup vs baseline: 15.4645x; 1.2794x over previous
import functools

import jax
import jax.numpy as jnp
from jax import lax
from jax.experimental import pallas as pl
from jax.experimental.pallas import tpu as pltpu
from jax.experimental.pallas import tpu_sc as plsc

VOCAB = 1000000
EMBED_DIM = 64
B = 16384
L = 50
N = B * L


TC_BS = 32768
TC_GRID = pl.cdiv(VOCAB, TC_BS)
VPAD = TC_GRID * TC_BS


def _precompute_body(tab_ref, w_ref, b_ref, out_ref):
    t = tab_ref[...]
    w = w_ref[...]
    zt = jnp.dot(w, t, preferred_element_type=jnp.float32)
    out_ref[...] = jax.nn.sigmoid(zt + b_ref[0, 0]).reshape(TC_BS)


def _precompute(table, W, b):
    t_t = table.T
    return pl.pallas_call(
        _precompute_body,
        grid=(TC_GRID,),
        in_specs=[
            pl.BlockSpec((EMBED_DIM, TC_BS), lambda i: (0, i)),
            pl.BlockSpec((1, EMBED_DIM), lambda i: (0, 0)),
            pl.BlockSpec(memory_space=pltpu.SMEM),
        ],
        out_specs=pl.BlockSpec((TC_BS,), lambda i: (i,)),
        out_shape=jax.ShapeDtypeStruct((VPAD,), jnp.float32),
    )(t_t, W.reshape(1, EMBED_DIM), b.reshape(1, 1))


_NC, _NS = 2, 16
_NW = _NC * _NS
_PER_W = N // _NW


def _gather_body(v_hbm, idx_hbm, out_hbm, idx_v, val_v, sem):
    wid = lax.axis_index("s") * _NC + lax.axis_index("c")
    base = wid * _PER_W
    pltpu.sync_copy(idx_hbm.at[pl.ds(base, _PER_W)], idx_v)
    pltpu.async_copy(v_hbm.at[idx_v], val_v, sem).wait()
    pltpu.sync_copy(val_v, out_hbm.at[pl.ds(base, _PER_W)])


def _sc_gather(v_flat, idx_flat):
    mesh = plsc.VectorSubcoreMesh(core_axis_name="c", subcore_axis_name="s")
    return pl.kernel(
        _gather_body,
        mesh=mesh,
        out_type=jax.ShapeDtypeStruct((N,), jnp.float32),
        scratch_types=[
            pltpu.VMEM((_PER_W,), jnp.int32),
            pltpu.VMEM((_PER_W,), jnp.float32),
            pltpu.SemaphoreType.DMA,
        ],
    )(v_flat, idx_flat)


def kernel(x, table, W, b):
    v = _precompute(table, W, b)
    idx = x.T.reshape(N).astype(jnp.int32)
    out = _sc_gather(v, idx)
    return out.reshape(L, B, 1).transpose((1, 0, 2))
